# baseline (speedup 1.0000x reference)
import functools
import math

import jax
import jax.numpy as jnp
import numpy as np
from jax import lax
from jax.experimental import pallas as pl
from jax.experimental.pallas import tpu as pltpu
from jax.experimental.pallas import tpu_sc as plsc

_B, _H, _L, _DK = 1, 12, 2048, 64
_C = _H * _DK


_NTAP = 26
_FIRST_TAPS = (0, 1, 2, 5, 8, 17)
_STAGE_ENDS = ((0, None), (1, 0), (4, None), (7, 1), (16, None), (25, 2))


def _conv_body(x_ref, gamma_ref, beta_ref, wproj_ref, w_ref,
               p_ref, km_ref, curp, acc):
  j = pl.program_id(1)

  @pl.when(j == 0)
  def _init():
    curp[...] = x_ref[0]

  r = jnp.where(j < 2, 0,
      jnp.where(j < 5, 3 - j,
      jnp.where(j < 8, 6 - j,
      jnp.where(j < 17, 12 - j, 21 - j))))
  rolled = pltpu.roll(curp[...], jnp.where(r < 0, r + _L, r), axis=1)
  col = jax.lax.broadcasted_iota(jnp.int32, (_C, _L), 1)
  xs = jnp.where((col >= r) & (col < _L + r), rolled, 0.0)
  t = jnp.dot(w_ref[0, 0], xs, preferred_element_type=jnp.float32)
  first = ((j == 0) | (j == 1) | (j == 2) | (j == 5) | (j == 8) | (j == 17))

  @pl.when(first)
  def _set():
    acc[...] = t

  @pl.when(jnp.logical_not(first))
  def _add():
    acc[...] += t

  def finalize(side_idx):
    a = acc[...]
    mu = jnp.mean(a, axis=1, keepdims=True)
    m2 = jnp.mean(a * a, axis=1, keepdims=True)
    var = m2 - mu * mu
    scale = jax.lax.rsqrt(var + 1e-5) * gamma_ref[...]
    shift = beta_ref[...] - mu * scale
    yh = a * scale + shift
    z = jnp.where(yh > 0, yh, jnp.exp(yh) - 1.0)
    if side_idx is None:
      curp[...] = z
    else:
      p = jnp.dot(z, wproj_ref[0], preferred_element_type=jnp.float32)
      p_ref[0, side_idx] = p
      for hh in range(4):
        base = 192 * hh
        pm = (p[base:base + 64] + p[base + 64:base + 128] +
              p[base + 128:base + 192]) * (1.0 / 3.0)
        km_ref[0, side_idx, 64 * hh:64 * hh + 64] = pm

  for jend, side_idx in _STAGE_ENDS:
    @pl.when(j == jend)
    def _fin(side_idx=side_idx):
      finalize(side_idx)


_NEG = -3.0e38
_NPAD = 8192


def _sc_topk_body(qp_hbm, out_hbm, ts, ob):
  wid = lax.axis_index("s") * 2 + lax.axis_index("c")

  @pl.when(wid < _H)
  def _work():
    def padb(vi, c):
      ts[pl.ds(vi * 16, 16)] = jnp.full((16,), _NEG, jnp.float32)
      return c
    lax.fori_loop(0, 128, padb, 0)
    pltpu.sync_copy(qp_hbm.at[wid], ts.at[pl.ds(2048, 6144)])

    def isort(vi, c):
      neg = jnp.where((vi & 1) == 0, jnp.float32(1.0), jnp.float32(-1.0))
      v = ts[pl.ds(vi * 16, 16)] * neg
      ts[pl.ds(vi * 16, 16)] = jnp.sort(v) * neg
      return c
    lax.fori_loop(0, 512, isort, 0)

    for lk in range(5, 14):
      blk = 1 << lk
      dd = blk // 32
      while dd >= 1:
        ldd = dd.bit_length() - 1

        def pair(p, c, dd=dd, ldd=ldd, blk=blk):
          i = (p & (dd - 1)) + ((p >> ldd) << (ldd + 1))
          o1 = i * 16
          o2 = o1 + dd * 16
          a = ts[pl.ds(o1, 16)]
          b = ts[pl.ds(o2, 16)]
          asc = (o1 & blk) == 0
          lo = jnp.minimum(a, b)
          hi = jnp.maximum(a, b)
          ts[pl.ds(o1, 16)] = jnp.where(asc, lo, hi)
          ts[pl.ds(o2, 16)] = jnp.where(asc, hi, lo)
          return c
        lax.fori_loop(0, 256, pair, 0)
        dd //= 2

      def vsort(vi, c, blk=blk):
        neg = jnp.where(((vi * 16) & blk) == 0,
                        jnp.float32(1.0), jnp.float32(-1.0))
        v = ts[pl.ds(vi * 16, 16)] * neg
        ts[pl.ds(vi * 16, 16)] = jnp.sort(v) * neg
        return c
      lax.fori_loop(0, 512, vsort, 0)

    def emit(kv, c):
      v = ts[pl.ds(_NPAD - 16 - kv * 16, 16)]
      ob[pl.ds(kv * 16, 16)] = lax.rev(v, (0,))
      return c
    lax.fori_loop(0, 128, emit, 0)
    pltpu.sync_copy(ob, out_hbm.at[wid])


def _sc_topk(qp):
  mesh = plsc.VectorSubcoreMesh(core_axis_name="c", subcore_axis_name="s")
  return pl.kernel(
      _sc_topk_body,
      out_type=jax.ShapeDtypeStruct((_H, _L), jnp.float32),
      mesh=mesh,
      scratch_types=[
          pltpu.VMEM((_NPAD,), jnp.float32),
          pltpu.VMEM((_L,), jnp.float32),
      ],
      compiler_params=pltpu.CompilerParams(needs_layout_passes=False),
  )(qp)


def _attn_body(qt_ref, km_ref, v_ref, pbq_ref, pbk_ref, attn_ref, ctx_ref):
  s = jnp.sum(pbq_ref[...] * pbk_ref[...], keepdims=True) * 0.125
  a = qt_ref[0] * s
  m = km_ref[0]
  logits = a * m
  mx = jnp.max(logits, axis=1, keepdims=True)
  e = jnp.exp(logits - mx)
  ssum = jnp.sum(e, axis=1, keepdims=True)
  attn = e / ssum
  attn_ref[0] = attn
  ctx_ref[0] = jnp.dot(attn, v_ref[0], preferred_element_type=jnp.float32)


def kernel(Q, K, V, attn_mask, wq0, bq0, wq1, bq1, wq2, bq2,
           wk0, bk0, wk1, bk1, wk2, bk2, gamma, beta,
           proj_q_w, proj_k_w, proj_back_q_w, proj_back_k_w):
  b, h, l, d_k = Q.shape
  del attn_mask

  x0 = jnp.stack([Q.reshape(_C, _L), K.reshape(_C, _L)])

  def prep_w(wq, wk):
    return jnp.stack([jnp.moveaxis(wq, 2, 0), jnp.moveaxis(wk, 2, 0)])

  w1 = prep_w(wq0, wk0)
  w3 = prep_w(wq1, wk1)
  w9 = prep_w(wq2, wk2)
  wcat = jnp.concatenate([w1, w1, w3, w3, w9, w9], axis=1)
  del bq0, bq1, bq2, bk0, bk1, bk2

  eye = jnp.eye(32, dtype=jnp.float32)
  def band(w):
    m = eye[:, None, :] * w[0][None, :, None]
    return m.reshape(2048, 32)
  wproj = jnp.stack([band(proj_q_w), band(proj_k_w)])

  p, km = pl.pallas_call(
      _conv_body,
      grid=(2, _NTAP),
      in_specs=[
          pl.BlockSpec((1, _C, _L), lambda i, j: (i, 0, 0)),
          pl.BlockSpec((_C, 1), lambda i, j: (0, 0)),
          pl.BlockSpec((_C, 1), lambda i, j: (0, 0)),
          pl.BlockSpec((1, 2048, 32), lambda i, j: (i, 0, 0)),
          pl.BlockSpec((1, 1, _C, _C), lambda i, j: (i, j, 0, 0)),
      ],
      out_specs=[
          pl.BlockSpec((1, 3, _C, 32), lambda i, j: (i, 0, 0, 0)),
          pl.BlockSpec((1, 3, 256, 32), lambda i, j: (i, 0, 0, 0)),
      ],
      out_shape=[
          jax.ShapeDtypeStruct((2, 3, _C, 32), jnp.float32),
          jax.ShapeDtypeStruct((2, 3, 256, 32), jnp.float32),
      ],
      scratch_shapes=[
          pltpu.VMEM((_C, _L), jnp.float32),
          pltpu.VMEM((_C, _L), jnp.float32),
      ],
      compiler_params=pltpu.CompilerParams(
          dimension_semantics=("arbitrary", "arbitrary")),
  )(x0, gamma[:, None], beta[:, None], wproj, wcat)

  qp = p[0].reshape(_H, 3 * _L)
  k_mean = km[1].reshape(_H, _L)

  q_top = _sc_topk(qp)

  qb = 256
  attn, ctx = pl.pallas_call(
      _attn_body,
      grid=(_H, _L // qb),
      in_specs=[
          pl.BlockSpec((1, qb, 1), lambda hh, j: (hh, j, 0)),
          pl.BlockSpec((1, 1, _L), lambda hh, j: (hh, 0, 0)),
          pl.BlockSpec((1, _L, _DK), lambda hh, j: (hh, 0, 0)),
          pl.BlockSpec((1, _DK), lambda hh, j: (0, 0)),
          pl.BlockSpec((1, _DK), lambda hh, j: (0, 0)),
      ],
      out_specs=[
          pl.BlockSpec((1, qb, _L), lambda hh, j: (hh, j, 0)),
          pl.BlockSpec((1, qb, _DK), lambda hh, j: (hh, j, 0)),
      ],
      out_shape=[
          jax.ShapeDtypeStruct((_H, _L, _L), jnp.float32),
          jax.ShapeDtypeStruct((_H, _L, _DK), jnp.float32),
      ],
      compiler_params=pltpu.CompilerParams(
          dimension_semantics=("arbitrary", "arbitrary")),
  )(q_top[:, :, None], k_mean[:, None, :], V[0],
    proj_back_q_w.reshape(1, _DK), proj_back_k_w.reshape(1, _DK))

  return (ctx.reshape(b, h, l, d_k), attn.reshape(b, h, l, l))


# speedup vs baseline: 1.2737x; 1.2737x over previous
import functools
import math

import jax
import jax.numpy as jnp
import numpy as np
from jax import lax
from jax.experimental import pallas as pl
from jax.experimental.pallas import tpu as pltpu
from jax.experimental.pallas import tpu_sc as plsc

_B, _H, _L, _DK = 1, 12, 2048, 64
_C = _H * _DK


_NTAP = 26
_FIRST_TAPS = (0, 1, 2, 5, 8, 17)
_STAGE_ENDS = ((0, None), (1, 0), (4, None), (7, 1), (16, None), (25, 2))


def _conv_body(x_ref, gamma_ref, beta_ref, wproj_ref, w_ref,
               p_ref, km_ref, curp, acc):
  j = pl.program_id(0)

  @pl.when(j == 0)
  def _init():
    curp[...] = x_ref[...]

  r = jnp.where(j < 2, 0,
      jnp.where(j < 5, 3 - j,
      jnp.where(j < 8, 6 - j,
      jnp.where(j < 17, 12 - j, 21 - j))))
  rolled = pltpu.roll(curp[...], jnp.where(r < 0, r + _L, r), axis=1)
  col = jax.lax.broadcasted_iota(jnp.int32, (_C, _L), 1)
  xs = jnp.where((col >= r) & (col < _L + r), rolled, 0.0)
  t = jnp.dot(w_ref[0], xs, preferred_element_type=jnp.float32)
  first = ((j == 0) | (j == 1) | (j == 2) | (j == 5) | (j == 8) | (j == 17))

  @pl.when(first)
  def _set():
    acc[...] = t

  @pl.when(jnp.logical_not(first))
  def _add():
    acc[...] += t

  def finalize(side_idx):
    a = acc[...]
    mu = jnp.mean(a, axis=1, keepdims=True)
    m2 = jnp.mean(a * a, axis=1, keepdims=True)
    var = m2 - mu * mu
    scale = jax.lax.rsqrt(var + 1e-5) * gamma_ref[...]
    shift = beta_ref[...] - mu * scale
    yh = a * scale + shift
    z = jnp.where(yh > 0, yh, jnp.exp(yh) - 1.0)
    if side_idx is None:
      curp[...] = z
    else:
      p = jnp.dot(z, wproj_ref[...], preferred_element_type=jnp.float32)
      p_ref[side_idx] = p
      for hh in range(4):
        base = 192 * hh
        pm = (p[base:base + 64] + p[base + 64:base + 128] +
              p[base + 128:base + 192]) * (1.0 / 3.0)
        km_ref[side_idx, 64 * hh:64 * hh + 64] = pm

  for jend, side_idx in _STAGE_ENDS:
    @pl.when(j == jend)
    def _fin(side_idx=side_idx):
      finalize(side_idx)


_NEG = -3.0e38
_NPAD = 8192


def _sc_topk_body(qp_hbm, out_hbm, ts, ob):
  wid = lax.axis_index("s") * 2 + lax.axis_index("c")

  @pl.when(wid < _H)
  def _work():
    def padb(vi, c):
      ts[pl.ds(vi * 16, 16)] = jnp.full((16,), _NEG, jnp.float32)
      return c
    lax.fori_loop(0, 128, padb, 0)
    pltpu.sync_copy(qp_hbm.at[wid], ts.at[pl.ds(2048, 6144)])

    def isort(vi, c):
      neg = jnp.where((vi & 1) == 0, jnp.float32(1.0), jnp.float32(-1.0))
      v = ts[pl.ds(vi * 16, 16)] * neg
      ts[pl.ds(vi * 16, 16)] = jnp.sort(v) * neg
      return c
    lax.fori_loop(0, 512, isort, 0)

    for lk in range(5, 14):
      blk = 1 << lk
      dd = blk // 32
      while dd >= 1:
        ldd = dd.bit_length() - 1

        def pair(p, c, dd=dd, ldd=ldd, blk=blk):
          i = (p & (dd - 1)) + ((p >> ldd) << (ldd + 1))
          o1 = i * 16
          o2 = o1 + dd * 16
          a = ts[pl.ds(o1, 16)]
          b = ts[pl.ds(o2, 16)]
          asc = (o1 & blk) == 0
          lo = jnp.minimum(a, b)
          hi = jnp.maximum(a, b)
          ts[pl.ds(o1, 16)] = jnp.where(asc, lo, hi)
          ts[pl.ds(o2, 16)] = jnp.where(asc, hi, lo)
          return c
        lax.fori_loop(0, 256, pair, 0)
        dd //= 2

      def vsort(vi, c, blk=blk):
        neg = jnp.where(((vi * 16) & blk) == 0,
                        jnp.float32(1.0), jnp.float32(-1.0))
        v = ts[pl.ds(vi * 16, 16)] * neg
        ts[pl.ds(vi * 16, 16)] = jnp.sort(v) * neg
        return c
      lax.fori_loop(0, 512, vsort, 0)

    def emit(kv, c):
      v = ts[pl.ds(_NPAD - 16 - kv * 16, 16)]
      ob[pl.ds(kv * 16, 16)] = lax.rev(v, (0,))
      return c
    lax.fori_loop(0, 128, emit, 0)
    pltpu.sync_copy(ob, out_hbm.at[wid])


def _sc_topk(qp):
  mesh = plsc.VectorSubcoreMesh(core_axis_name="c", subcore_axis_name="s")
  return pl.kernel(
      _sc_topk_body,
      out_type=jax.ShapeDtypeStruct((_H, _L), jnp.float32),
      mesh=mesh,
      scratch_types=[
          pltpu.VMEM((_NPAD,), jnp.float32),
          pltpu.VMEM((_L,), jnp.float32),
      ],
      compiler_params=pltpu.CompilerParams(needs_layout_passes=False),
  )(qp)


def _attn_body(qt_ref, km_ref, v_ref, pbq_ref, pbk_ref, attn_ref, ctx_ref):
  s = jnp.sum(pbq_ref[...] * pbk_ref[...], keepdims=True) * 0.125
  a = qt_ref[0] * s
  m = km_ref[0]
  logits = a * m
  mx = jnp.max(logits, axis=1, keepdims=True)
  e = jnp.exp(logits - mx)
  ssum = jnp.sum(e, axis=1, keepdims=True)
  attn = e / ssum
  attn_ref[0] = attn
  ctx_ref[0] = jnp.dot(attn, v_ref[0], preferred_element_type=jnp.float32)


def kernel(Q, K, V, attn_mask, wq0, bq0, wq1, bq1, wq2, bq2,
           wk0, bk0, wk1, bk1, wk2, bk2, gamma, beta,
           proj_q_w, proj_k_w, proj_back_q_w, proj_back_k_w):
  b, h, l, d_k = Q.shape
  del attn_mask

  def prep_w(w0, w1, w2):
    a = jnp.moveaxis(w0, 2, 0)
    b = jnp.moveaxis(w1, 2, 0)
    c = jnp.moveaxis(w2, 2, 0)
    return jnp.concatenate([a, a, b, b, c, c], axis=0)

  del bq0, bq1, bq2, bk0, bk1, bk2

  eye = jnp.eye(32, dtype=jnp.float32)
  def band(w):
    m = eye[:, None, :] * w[0][None, :, None]
    return m.reshape(2048, 32)

  conv_call = pl.pallas_call(
      _conv_body,
      grid=(_NTAP,),
      in_specs=[
          pl.BlockSpec((_C, _L), lambda j: (0, 0)),
          pl.BlockSpec((_C, 1), lambda j: (0, 0)),
          pl.BlockSpec((_C, 1), lambda j: (0, 0)),
          pl.BlockSpec((2048, 32), lambda j: (0, 0)),
          pl.BlockSpec((1, _C, _C), lambda j: (j, 0, 0)),
      ],
      out_specs=[
          pl.BlockSpec((3, _C, 32), lambda j: (0, 0, 0)),
          pl.BlockSpec((3, 256, 32), lambda j: (0, 0, 0)),
      ],
      out_shape=[
          jax.ShapeDtypeStruct((3, _C, 32), jnp.float32),
          jax.ShapeDtypeStruct((3, 256, 32), jnp.float32),
      ],
      scratch_shapes=[
          pltpu.VMEM((_C, _L), jnp.float32),
          pltpu.VMEM((_C, _L), jnp.float32),
      ],
      compiler_params=pltpu.CompilerParams(
          dimension_semantics=("arbitrary",)),
  )

  g2 = gamma[:, None]
  b2 = beta[:, None]
  p_q, _ = conv_call(Q.reshape(_C, _L), g2, b2, band(proj_q_w),
                     prep_w(wq0, wq1, wq2))
  qp = p_q.reshape(_H, 3 * _L)
  q_top = _sc_topk(qp)

  _, km_k = conv_call(K.reshape(_C, _L), g2, b2, band(proj_k_w),
                      prep_w(wk0, wk1, wk2))
  k_mean = km_k.reshape(_H, _L)

  qb = 256
  attn, ctx = pl.pallas_call(
      _attn_body,
      grid=(_H, _L // qb),
      in_specs=[
          pl.BlockSpec((1, qb, 1), lambda hh, j: (hh, j, 0)),
          pl.BlockSpec((1, 1, _L), lambda hh, j: (hh, 0, 0)),
          pl.BlockSpec((1, _L, _DK), lambda hh, j: (hh, 0, 0)),
          pl.BlockSpec((1, _DK), lambda hh, j: (0, 0)),
          pl.BlockSpec((1, _DK), lambda hh, j: (0, 0)),
      ],
      out_specs=[
          pl.BlockSpec((1, qb, _L), lambda hh, j: (hh, j, 0)),
          pl.BlockSpec((1, qb, _DK), lambda hh, j: (hh, j, 0)),
      ],
      out_shape=[
          jax.ShapeDtypeStruct((_H, _L, _L), jnp.float32),
          jax.ShapeDtypeStruct((_H, _L, _DK), jnp.float32),
      ],
      compiler_params=pltpu.CompilerParams(
          dimension_semantics=("arbitrary", "arbitrary")),
  )(q_top[:, :, None], k_mean[:, None, :], V[0],
    proj_back_q_w.reshape(1, _DK), proj_back_k_w.reshape(1, _DK))

  return (ctx.reshape(b, h, l, d_k), attn.reshape(b, h, l, l))
